# baseline (speedup 1.0000x reference)
import jax
import jax.numpy as jnp
from jax import lax
from jax.experimental import pallas as pl
from jax.experimental.pallas import tpu as pltpu
from jax.experimental.pallas import tpu_sc as plsc

VOCAB = 32000
HIDDEN = 4096
BATCH = 4
SEQ = 4096

NC = 2
NS = 16
NW = NC * NS
L = 16

B = BATCH * SEQ
B_PER_W = B // NW
K = 8
NBUF = 3
RA = 1
NCHUNK = B_PER_W // K


def _gather_body(inp_hbm, table_hbm, out_hbm, stage_v, idx_v, bufs,
                 gsems, psems):
    wid = lax.axis_index("s") * NC + lax.axis_index("c")
    base = wid * B_PER_W

    pltpu.sync_copy(inp_hbm.at[pl.ds(2 * base, 2 * B_PER_W)], stage_v)
    lane = lax.iota(jnp.int32, L)
    even = (lane * 2) % L
    lo_half = lane < (L // 2)

    dnums = lax.GatherDimensionNumbers(
        offset_dims=(), collapsed_slice_dims=(0,), start_index_map=(0,))

    def lane_gather(x, idx):
        return lax.gather(x, idx[:, None], dnums, slice_sizes=(1,),
                          mode=lax.GatherScatterMode.PROMISE_IN_BOUNDS)

    def destride(v):
        a = stage_v[pl.ds(2 * L * v, L)]
        b = stage_v[pl.ds(2 * L * v + L, L)]
        idx_v[pl.ds(L * v, L)] = jnp.where(
            lo_half, lane_gather(a, even), lane_gather(b, even))

    def gather_start(g, b):
        pltpu.async_copy(table_hbm.at[idx_v.at[pl.ds(g * K, K)]], bufs[b],
                         gsems[b])

    def gather_wait(b):
        pltpu.make_async_copy(table_hbm.at[idx_v.at[pl.ds(0, K)]], bufs[b],
                              gsems[b]).wait()

    def put_start(g, b):
        pltpu.async_copy(bufs[b], out_hbm.at[pl.ds(base + g * K, K)], psems[b])

    def put_wait(b):
        pltpu.make_async_copy(bufs[b], out_hbm.at[pl.ds(base, K)],
                              psems[b]).wait()

    first = max(1, (RA * K + L - 1) // L)
    for v in range(first):
        destride(v)
    for g in range(RA):
        gather_start(g, g % NBUF)
    for v in range(first, B_PER_W // L):
        destride(v)


    def iter_step(i, b, bnext, do_putwait, do_gather):
        if do_gather:
            if do_putwait:
                put_wait(bnext)
            gather_start(i + RA, bnext)
        gather_wait(b)
        put_start(i, b)

    H = NBUF - RA
    BULK = ((NCHUNK - RA) - H) // NBUF * NBUF
    for i in range(H):
        iter_step(i, i % NBUF, (i + RA) % NBUF, False, True)

    @pl.loop(H, H + BULK, step=NBUF)
    def _(i0):
        for j in range(NBUF):
            iter_step(i0 + j, (H + j) % NBUF, (H + j + RA) % NBUF, True, True)

    for i in range(H + BULK, NCHUNK - RA):
        iter_step(i, i % NBUF, (i + RA) % NBUF, True, True)
    for i in range(NCHUNK - RA, NCHUNK):
        iter_step(i, i % NBUF, None, False, False)
    for g in range(NCHUNK - NBUF, NCHUNK):
        put_wait(g % NBUF)


@jax.jit
def _embed_gather(inp_flat, W):
    mesh = plsc.VectorSubcoreMesh(core_axis_name="c", subcore_axis_name="s")
    run = pl.kernel(
        _gather_body,
        out_type=jax.ShapeDtypeStruct((B, HIDDEN), jnp.float32),
        mesh=mesh,
        scratch_types=[
            pltpu.VMEM((2 * B_PER_W,), jnp.int32),
            pltpu.VMEM((B_PER_W,), jnp.int32),
            [pltpu.VMEM((K, HIDDEN), jnp.float32) for _ in range(NBUF)],
            [pltpu.SemaphoreType.DMA for _ in range(NBUF)],
            [pltpu.SemaphoreType.DMA for _ in range(NBUF)],
        ],
    )
    return run(inp_flat, W)


def kernel(inputs, W):
    hidden = _embed_gather(inputs.reshape(2 * B), W)
    return hidden.reshape(BATCH, SEQ, HIDDEN), inputs[..., 1]


# speedup vs baseline: 1.0578x; 1.0578x over previous
import jax
import jax.numpy as jnp
from jax import lax
from jax.experimental import pallas as pl
from jax.experimental.pallas import tpu as pltpu
from jax.experimental.pallas import tpu_sc as plsc

VOCAB = 32000
HIDDEN = 4096
BATCH = 4
SEQ = 4096

NC = 2
NS = 16
NW = NC * NS

B = BATCH * SEQ
B_PER_W = B // NW
K = 8
NBUF = 2
RA = 1
NCHUNK = B_PER_W // K


def _gather_body(ids_hbm, table_hbm, out_hbm, idx_v, bufs, gsems, psems):
    wid = lax.axis_index("s") * NC + lax.axis_index("c")
    base = wid * B_PER_W

    pltpu.sync_copy(ids_hbm.at[pl.ds(base, B_PER_W)], idx_v)

    def gather_start(g, b):
        pltpu.async_copy(table_hbm.at[idx_v.at[pl.ds(g * K, K)]], bufs[b],
                         gsems[b])

    def gather_wait(b):
        pltpu.make_async_copy(table_hbm.at[idx_v.at[pl.ds(0, K)]], bufs[b],
                              gsems[b]).wait()

    def put_start(g, b):
        pltpu.async_copy(bufs[b], out_hbm.at[pl.ds(base + g * K, K)], psems[b])

    def put_wait(b):
        pltpu.make_async_copy(bufs[b], out_hbm.at[pl.ds(base, K)],
                              psems[b]).wait()


    def iter_step(i, b, bnext, do_putwait, do_gather):
        if do_gather:
            if do_putwait:
                put_wait(bnext)
            gather_start(i + RA, bnext)
        gather_wait(b)
        put_start(i, b)

    for g in range(RA):
        gather_start(g, g % NBUF)

    H = NBUF - RA
    BULK = ((NCHUNK - RA) - H) // NBUF * NBUF
    for i in range(H):
        iter_step(i, i % NBUF, (i + RA) % NBUF, False, True)

    @pl.loop(H, H + BULK, step=NBUF)
    def _(i0):
        for j in range(NBUF):
            iter_step(i0 + j, (H + j) % NBUF, (H + j + RA) % NBUF, True, True)

    for i in range(H + BULK, NCHUNK - RA):
        iter_step(i, i % NBUF, (i + RA) % NBUF, True, True)
    for i in range(NCHUNK - RA, NCHUNK):
        iter_step(i, i % NBUF, None, False, False)
    for g in range(NCHUNK - NBUF, NCHUNK):
        put_wait(g % NBUF)


@jax.jit
def _embed_gather(ids, W):
    mesh = plsc.VectorSubcoreMesh(core_axis_name="c", subcore_axis_name="s")
    run = pl.kernel(
        _gather_body,
        out_type=jax.ShapeDtypeStruct((B, HIDDEN), jnp.float32),
        mesh=mesh,
        scratch_types=[
            pltpu.VMEM((B_PER_W,), jnp.int32),
            [pltpu.VMEM((K, HIDDEN), jnp.float32) for _ in range(NBUF)],
            [pltpu.SemaphoreType.DMA for _ in range(NBUF)],
            [pltpu.SemaphoreType.DMA for _ in range(NBUF)],
        ],
    )
    return run(ids, W)


def kernel(inputs, W):
    ids = inputs[..., 0].reshape(B)
    attention_mask = inputs[..., 1]
    hidden = _embed_gather(ids, W)
    return hidden.reshape(BATCH, SEQ, HIDDEN), attention_mask
